# baseline (speedup 1.0000x reference)
import functools

import jax
import jax.numpy as jnp
from jax import lax
from jax.experimental import pallas as pl
from jax.experimental.pallas import tpu as pltpu
from jax.experimental.pallas import tpu_sc as plsc

N = 10000
E = 160000
D = 256
HALF = 128

NSUB = 16
EPT = E // NSUB
EPT_PAD = 10240
CH = 128
NCH = EPT_PAD // CH
HC = (NCH + 2) // 2
NPAD = 10240
NPT = NPAD // NSUB

_MESH = plsc.VectorSubcoreMesh(core_axis_name="c", subcore_axis_name="s")


@functools.partial(
    pl.kernel,
    mesh=_MESH,
    out_type=jax.ShapeDtypeStruct((2 * NPAD, HALF), jnp.float32),
    scratch_types=[
        pltpu.VMEM((2, CH), jnp.int32),
        pltpu.VMEM((2, CH), jnp.int32),
        pltpu.VMEM((CH, HALF), jnp.float32),
        pltpu.VMEM((CH, HALF), jnp.float32),
        pltpu.VMEM_SHARED((NPAD, HALF), jnp.float32),
        pltpu.SemaphoreType.DMA,
        pltpu.SemaphoreType.DMA,
        pltpu.SemaphoreType.DMA,
        pltpu.SemaphoreType.DMA,
    ],
)
def _segsum(
    x_hbm, e_hbm, z_hbm, out_hbm,
    i0, i1, r0, r1, acc, gi0, gi1, gr0, gr1,
):
    c = lax.axis_index("c")
    s = lax.axis_index("s")
    pltpu.sync_copy(z_hbm, acc.at[pl.ds(s * NPT, NPT)])
    plsc.subcore_barrier()

    def fire_idx(j, ib, sem):
        pltpu.async_copy(e_hbm.at[c, s, j], ib, sem)

    def wait_idx(ib, sem):
        pltpu.make_async_copy(e_hbm.at[c, s, 0], ib, sem).wait()

    def fire_gather(ib, rb, sem):
        pltpu.async_copy(x_hbm.at[ib.at[0]], rb, sem)

    def wait_gather(ib, rb, sem):
        pltpu.make_async_copy(x_hbm.at[ib.at[0]], rb, sem).wait()

    def scatter(ib, rb):
        pltpu.sync_copy(rb, acc.at[ib.at[1]], add=True)

    fire_idx(0, i0, gi0)
    fire_idx(1, i1, gi1)
    wait_idx(i0, gi0)
    fire_gather(i0, r0, gr0)

    def step(j, ia, ra, gia, gra, ib, rb, gib, grb):
        wait_gather(ia, ra, gra)
        wait_idx(ib, gib)
        fire_gather(ib, rb, grb)
        scatter(ia, ra)
        fire_idx(j + 2, ia, gia)

    def pair(p, _):
        step(2 * p, i0, r0, gi0, gr0, i1, r1, gi1, gr1)
        step(2 * p + 1, i1, r1, gi1, gr1, i0, r0, gi0, gr0)
        return 0

    lax.fori_loop(0, (NCH - 2) // 2, pair, 0)
    step(NCH - 2, i0, r0, gi0, gr0, i1, r1, gi1, gr1)
    wait_gather(i1, r1, gr1)
    scatter(i1, r1)
    wait_idx(i0, gi0)
    plsc.subcore_barrier()
    pltpu.sync_copy(
        acc.at[pl.ds(s * NPT, NPT)], out_hbm.at[pl.ds(c * NPAD + s * NPT, NPT)]
    )


@functools.partial(
    pl.kernel,
    mesh=_MESH,
    out_type=jax.ShapeDtypeStruct((2 * NPAD, HALF), jnp.float32),
    scratch_types=[
        pltpu.VMEM((2 * HC, CH), jnp.int32),
        pltpu.VMEM((CH, HALF), jnp.float32),
        pltpu.VMEM_SHARED((NPAD, HALF), jnp.float32),
    ],
)
def _degree(dst_hbm, z_hbm, ones_hbm, out_hbm, didx, ones_v, acc):
    c = lax.axis_index("c")
    s = lax.axis_index("s")
    pltpu.sync_copy(z_hbm, acc.at[pl.ds(s * NPT, NPT)])
    pltpu.sync_copy(dst_hbm.at[s], didx)
    pltpu.sync_copy(ones_hbm, ones_v)
    plsc.subcore_barrier()
    base = c * HC

    def chunk(j, _):
        pltpu.sync_copy(ones_v, acc.at[didx.at[base + j]], add=True)
        return 0

    lax.fori_loop(0, HC, chunk, 0)
    plsc.subcore_barrier()
    pltpu.sync_copy(
        acc.at[pl.ds(s * NPT, NPT)], out_hbm.at[pl.ds(c * NPAD + s * NPT, NPT)]
    )


def _tc_layer(a, xs, deg2, Wl, Wr, b2, relu, split):
    R = 1000
    grid = (N // R,)

    def body(a_ref, x_ref, d_ref, wl_ref, wr_ref, b_ref, o_ref):
        d = d_ref[0][:, 0:1] + d_ref[1][:, 0:1]
        rd = 1.0 / jnp.maximum(d, 1.0)
        agg = jnp.concatenate([a_ref[0], a_ref[1]], axis=1) * rd
        xx = jnp.concatenate([x_ref[0], x_ref[1]], axis=1)
        h = (
            jnp.dot(agg, wl_ref[...], preferred_element_type=jnp.float32)
            + jnp.dot(xx, wr_ref[...], preferred_element_type=jnp.float32)
            + b_ref[...]
        )
        if relu:
            h = jnp.maximum(h, 0.0)
        if split:
            o_ref[0] = h[:, :HALF]
            o_ref[1] = h[:, HALF:]
        else:
            o_ref[...] = h

    in_specs = [
        pl.BlockSpec((2, R, HALF), lambda i: (0, i, 0)),
        pl.BlockSpec((2, R, HALF), lambda i: (0, i, 0)),
        pl.BlockSpec((2, R, HALF), lambda i: (0, i, 0)),
        pl.BlockSpec((D, D), lambda i: (0, 0)),
        pl.BlockSpec((D, D), lambda i: (0, 0)),
        pl.BlockSpec((1, D), lambda i: (0, 0)),
    ]
    if split:
        out_shape = jax.ShapeDtypeStruct((2, N, HALF), jnp.float32)
        out_spec = pl.BlockSpec((2, R, HALF), lambda i: (0, i, 0))
    else:
        out_shape = jax.ShapeDtypeStruct((N, D), jnp.float32)
        out_spec = pl.BlockSpec((R, D), lambda i: (i, 0))
    return pl.pallas_call(
        body,
        grid=grid,
        in_specs=in_specs,
        out_specs=out_spec,
        out_shape=out_shape,
    )(a, xs, deg2, Wl, Wr, b2)


def kernel(x, adj_t, Wl1, Wr1, b1, Wl2, Wr2, b2, Wl3, Wr3, b3):
    src = adj_t[0].astype(jnp.int32)
    dst = adj_t[1].astype(jnp.int32)
    src2 = jnp.pad(src.reshape(NSUB, EPT), ((0, 0), (0, EPT_PAD - EPT)))
    dst2 = jnp.pad(
        dst.reshape(NSUB, EPT), ((0, 0), (0, EPT_PAD - EPT)), constant_values=N
    )
    src3 = src2.reshape(NSUB, NCH, CH)
    dst3 = dst2.reshape(NSUB, NCH, CH)
    e = jnp.stack([
        jnp.stack([src3, dst3], axis=2),
        jnp.stack([src3 + N, dst3], axis=2),
    ])
    e = jnp.pad(e, ((0, 0), (0, 0), (0, 1), (0, 0), (0, 0)))
    dst3d = jnp.pad(dst3, ((0, 0), (0, 2 * HC - NCH), (0, 0)), constant_values=N)
    zrows = jnp.zeros((NPT, HALF), jnp.float32)
    ones128 = jnp.ones((CH, HALF), jnp.float32)

    deg2 = _degree(dst3d, zrows, ones128).reshape(2, NPAD, HALF)

    xs = jnp.stack([x[:, :HALF], x[:, HALF:]])
    h = xs
    layers = [(Wl1, Wr1, b1), (Wl2, Wr2, b2), (Wl3, Wr3, b3)]
    for li, (Wl, Wr, b) in enumerate(layers):
        a = _segsum(h.reshape(2 * N, HALF), e, zrows)
        a = a.reshape(2, NPAD, HALF)
        last = li == 2
        h = _tc_layer(
            a, h, deg2, Wl, Wr, b.reshape(1, D), relu=not last, split=not last
        )
    return h


# speedup vs baseline: 1.6876x; 1.6876x over previous
import functools

import jax
import jax.numpy as jnp
from jax import lax
from jax.experimental import pallas as pl
from jax.experimental.pallas import tpu as pltpu
from jax.experimental.pallas import tpu_sc as plsc

N = 10000
E = 160000
D = 256
HALF = 128

NSUB = 16
EPT = E // NSUB
EPT_PAD = 10000
CH = 80
NCH = EPT_PAD // CH
HC = (NCH + 2) // 2
NPAD = 10240
NPT = NPAD // NSUB

_MESH = plsc.VectorSubcoreMesh(core_axis_name="c", subcore_axis_name="s")


@functools.partial(
    pl.kernel,
    mesh=_MESH,
    out_type=jax.ShapeDtypeStruct((2 * NPAD, HALF), jnp.float32),
    scratch_types=[
        pltpu.VMEM((2, CH), jnp.int32),
        pltpu.VMEM((2, CH), jnp.int32),
        pltpu.VMEM((2, CH), jnp.int32),
        pltpu.VMEM((CH, HALF), jnp.float32),
        pltpu.VMEM((CH, HALF), jnp.float32),
        pltpu.VMEM((CH, HALF), jnp.float32),
        pltpu.VMEM_SHARED((NPAD, HALF), jnp.float32),
        pltpu.SemaphoreType.DMA,
        pltpu.SemaphoreType.DMA,
        pltpu.SemaphoreType.DMA,
        pltpu.SemaphoreType.DMA,
        pltpu.SemaphoreType.DMA,
        pltpu.SemaphoreType.DMA,
    ],
)
def _segsum(
    x_hbm, e_hbm, z_hbm, out_hbm,
    i0, i1, i2, r0, r1, r2, acc, gi0, gi1, gi2, gr0, gr1, gr2,
):
    c = lax.axis_index("c")
    s = lax.axis_index("s")
    pltpu.sync_copy(z_hbm, acc.at[pl.ds(s * NPT, NPT)])
    plsc.subcore_barrier()

    def fire_idx(j, ib, sem):
        pltpu.async_copy(e_hbm.at[c, s, j], ib, sem)

    def wait_idx(ib, sem):
        pltpu.make_async_copy(e_hbm.at[c, s, 0], ib, sem).wait()

    def fire_gather(ib, rb, sem):
        pltpu.async_copy(x_hbm.at[ib.at[0]], rb, sem)

    def wait_gather(ib, rb, sem):
        pltpu.make_async_copy(x_hbm.at[ib.at[0]], rb, sem).wait()

    def scatter(ib, rb):
        pltpu.sync_copy(rb, acc.at[ib.at[1]], add=True)

    fire_idx(0, i0, gi0)
    fire_idx(1, i1, gi1)
    fire_idx(2, i2, gi2)
    wait_idx(i0, gi0)
    fire_gather(i0, r0, gr0)
    wait_idx(i1, gi1)
    fire_gather(i1, r1, gr1)

    def step(j, a, b, cc):
        ia, ra, gia, gra = a
        ic, rc, gic, grc = cc
        wait_gather(ia, ra, gra)
        wait_idx(ic, gic)
        fire_gather(ic, rc, grc)
        scatter(ia, ra)
        fire_idx(j + 3, ia, gia)

    b0 = (i0, r0, gi0, gr0)
    b1 = (i1, r1, gi1, gr1)
    b2 = (i2, r2, gi2, gr2)

    def triple(p, _):
        step(3 * p, b0, b1, b2)
        step(3 * p + 1, b1, b2, b0)
        step(3 * p + 2, b2, b0, b1)
        return 0

    lax.fori_loop(0, (NCH - 2) // 3, triple, 0)
    wait_gather(i0, r0, gr0)
    scatter(i0, r0)
    wait_gather(i1, r1, gr1)
    scatter(i1, r1)
    wait_idx(i2, gi2)
    plsc.subcore_barrier()
    pltpu.sync_copy(
        acc.at[pl.ds(s * NPT, NPT)], out_hbm.at[pl.ds(c * NPAD + s * NPT, NPT)]
    )


@functools.partial(
    pl.kernel,
    mesh=_MESH,
    out_type=jax.ShapeDtypeStruct((2 * NPAD, HALF), jnp.float32),
    scratch_types=[
        pltpu.VMEM((2 * HC, CH), jnp.int32),
        pltpu.VMEM((CH, HALF), jnp.float32),
        pltpu.VMEM_SHARED((NPAD, HALF), jnp.float32),
    ],
)
def _degree(dst_hbm, z_hbm, ones_hbm, out_hbm, didx, ones_v, acc):
    c = lax.axis_index("c")
    s = lax.axis_index("s")
    pltpu.sync_copy(z_hbm, acc.at[pl.ds(s * NPT, NPT)])
    pltpu.sync_copy(dst_hbm.at[s], didx)
    pltpu.sync_copy(ones_hbm, ones_v)
    plsc.subcore_barrier()
    base = c * HC

    def chunk(j, _):
        pltpu.sync_copy(ones_v, acc.at[didx.at[base + j]], add=True)
        return 0

    lax.fori_loop(0, HC, chunk, 0)
    plsc.subcore_barrier()
    pltpu.sync_copy(
        acc.at[pl.ds(s * NPT, NPT)], out_hbm.at[pl.ds(c * NPAD + s * NPT, NPT)]
    )


def _tc_layer(a, xs, deg2, Wl, Wr, b2, relu, split):
    R = 1000
    grid = (N // R,)

    def body(a_ref, x_ref, d_ref, wl_ref, wr_ref, b_ref, o_ref):
        d = d_ref[0][:, 0:1] + d_ref[1][:, 0:1]
        rd = 1.0 / jnp.maximum(d, 1.0)
        agg = jnp.concatenate([a_ref[0], a_ref[1]], axis=1) * rd
        xx = jnp.concatenate([x_ref[0], x_ref[1]], axis=1)
        h = (
            jnp.dot(agg, wl_ref[...], preferred_element_type=jnp.float32)
            + jnp.dot(xx, wr_ref[...], preferred_element_type=jnp.float32)
            + b_ref[...]
        )
        if relu:
            h = jnp.maximum(h, 0.0)
        if split:
            o_ref[0] = h[:, :HALF]
            o_ref[1] = h[:, HALF:]
        else:
            o_ref[...] = h

    in_specs = [
        pl.BlockSpec((2, R, HALF), lambda i: (0, i, 0)),
        pl.BlockSpec((2, R, HALF), lambda i: (0, i, 0)),
        pl.BlockSpec((2, R, HALF), lambda i: (0, i, 0)),
        pl.BlockSpec((D, D), lambda i: (0, 0)),
        pl.BlockSpec((D, D), lambda i: (0, 0)),
        pl.BlockSpec((1, D), lambda i: (0, 0)),
    ]
    if split:
        out_shape = jax.ShapeDtypeStruct((2, N, HALF), jnp.float32)
        out_spec = pl.BlockSpec((2, R, HALF), lambda i: (0, i, 0))
    else:
        out_shape = jax.ShapeDtypeStruct((N, D), jnp.float32)
        out_spec = pl.BlockSpec((R, D), lambda i: (i, 0))
    return pl.pallas_call(
        body,
        grid=grid,
        in_specs=in_specs,
        out_specs=out_spec,
        out_shape=out_shape,
    )(a, xs, deg2, Wl, Wr, b2)


def kernel(x, adj_t, Wl1, Wr1, b1, Wl2, Wr2, b2, Wl3, Wr3, b3):
    src = adj_t[0].astype(jnp.int32)
    dst = adj_t[1].astype(jnp.int32)
    src2 = jnp.pad(src.reshape(NSUB, EPT), ((0, 0), (0, EPT_PAD - EPT)))
    dst2 = jnp.pad(
        dst.reshape(NSUB, EPT), ((0, 0), (0, EPT_PAD - EPT)), constant_values=N
    )
    src3 = src2.reshape(NSUB, NCH, CH)
    dst3 = dst2.reshape(NSUB, NCH, CH)
    e = jnp.stack([
        jnp.stack([src3, dst3], axis=2),
        jnp.stack([src3 + N, dst3], axis=2),
    ])
    e = jnp.pad(e, ((0, 0), (0, 0), (0, 1), (0, 0), (0, 0)))
    dst3d = jnp.pad(dst3, ((0, 0), (0, 2 * HC - NCH), (0, 0)), constant_values=N)
    zrows = jnp.zeros((NPT, HALF), jnp.float32)
    ones128 = jnp.ones((CH, HALF), jnp.float32)

    deg2 = _degree(dst3d, zrows, ones128).reshape(2, NPAD, HALF)

    xs = jnp.stack([x[:, :HALF], x[:, HALF:]])
    h = xs
    layers = [(Wl1, Wr1, b1), (Wl2, Wr2, b2), (Wl3, Wr3, b3)]
    for li, (Wl, Wr, b) in enumerate(layers):
        a = _segsum(h.reshape(2 * N, HALF), e, zrows)
        a = a.reshape(2, NPAD, HALF)
        last = li == 2
        h = _tc_layer(
            a, h, deg2, Wl, Wr, b.reshape(1, D), relu=not last, split=not last
        )
    return h
